# baseline (speedup 1.0000x reference)
import jax
import jax.numpy as jnp
from jax.experimental import pallas as pl

EMB = 1000
ROWS_PER_BLOCK = 1024


def _onehot_block(labels_ref, out_ref):
    labels = labels_ref[:].astype(jnp.int32)
    cols = jax.lax.broadcasted_iota(jnp.int32, (ROWS_PER_BLOCK, EMB), 1)
    out_ref[:, :] = (labels[:, None] == cols).astype(jnp.float32)


def kernel(labels):
    batch = labels.shape[0]
    grid = batch // ROWS_PER_BLOCK
    return pl.pallas_call(
        _onehot_block,
        grid=(grid,),
        in_specs=[pl.BlockSpec((ROWS_PER_BLOCK,), lambda i: (i,))],
        out_specs=pl.BlockSpec((ROWS_PER_BLOCK, EMB), lambda i: (i, 0)),
        out_shape=jax.ShapeDtypeStruct((batch, EMB), jnp.float32),
    )(labels)
